# baseline (speedup 1.0000x reference)
import functools

import jax
import jax.numpy as jnp
from jax.experimental import pallas as pl
from jax.experimental.pallas import tpu as pltpu
from jax.experimental.pallas import tpu_sc as plsc

_SMOOTH = 0.1
_PAD = 0
_CONF = 1.0 - _SMOOTH

_NSTRIPE = 8
_RB = 32
_W = 12544

_SC_ROWS = 512
_CW = 3840
_NCH = 26


def _tc_kernel(*refs, C):
    xrefs = refs[:_NSTRIPE]
    y_ref = refs[_NSTRIPE]
    out_ref = refs[_NSTRIPE + 1]
    yb = y_ref[...]

    ms, ss, ts, gs = [], [], [], []
    x0 = None
    for q, xr in enumerate(xrefs):
        xq = xr[...]
        cols = jax.lax.broadcasted_iota(jnp.int32, xq.shape, 1)
        n_valid = C - q * _W
        if n_valid < _W:
            xv = jnp.where(cols < n_valid, xq, -jnp.inf)
            tq = jnp.sum(jnp.where(cols < n_valid, xq, 0.0), axis=1,
                         keepdims=True)
        else:
            xv = xq
            tq = jnp.sum(xq, axis=1, keepdims=True)
        mq = jnp.max(xv, axis=1, keepdims=True)
        ms.append(mq)
        ss.append(jnp.sum(jnp.exp(xv - mq), axis=1, keepdims=True))
        ts.append(tq)
        gs.append(jnp.sum(jnp.where(cols == yb - q * _W, xq, 0.0), axis=1,
                          keepdims=True))
        if q == 0:
            x0 = xq[:, 0:1]

    m = functools.reduce(jnp.maximum, ms)
    s = sum(sq * jnp.exp(mq - m) for sq, mq in zip(ss, ms))
    t = sum(ts)
    g = sum(gs)

    eps = _SMOOTH / (C - 2)
    K = _SMOOTH * jnp.log(eps) + _CONF * jnp.log(_CONF)
    lse = m + jnp.log(s)
    ssum = t - C * lse
    logp0 = x0 - lse
    logpy = g - lse
    row = K - eps * (ssum - logp0 - logpy) - _CONF * logpy
    out_ref[...] = jnp.where(yb != _PAD, row, 0.0)


def _corner_kernel(x_ref, y_ref, m_ref, s_ref, t_ref, g_ref, *, CW, col0):
    xq = x_ref[...]
    yb = y_ref[...]
    cols = jax.lax.broadcasted_iota(jnp.int32, xq.shape, 1)
    valid = cols < CW
    xv = jnp.where(valid, xq, -jnp.inf)
    mq = jnp.max(xv, axis=1, keepdims=True)
    m_ref[...] = mq
    s_ref[...] = jnp.sum(jnp.exp(xv - mq), axis=1, keepdims=True)
    t_ref[...] = jnp.sum(jnp.where(valid, xq, 0.0), axis=1, keepdims=True)
    g_ref[...] = jnp.sum(jnp.where(cols == yb - col0, xq, 0.0), axis=1,
                         keepdims=True)


def _sc_kernel(x_hbm, y_hbm, o_hbm, m_s, s_s, t_s, g_s, x0_s, *, row0):
    iota = jax.lax.iota(jnp.int32, 16)

    def chunk_body(x_vmem, y_vmem, o_vmem):
        c = pl.program_id(1)
        for r8 in range(8):
            ch = x_vmem.at[r8]
            y0v = y_vmem.at[r8][pl.ds(0, 16)]
            mrow = m_s.at[r8]
            srow = s_s.at[r8]
            trow = t_s.at[r8]
            grow = g_s.at[r8]
            x0row = x0_s.at[r8]

            @pl.when(c == 0)
            def _init():
                mrow[...] = jnp.full((16,), -jnp.inf, jnp.float32)
                srow[...] = jnp.zeros((16,), jnp.float32)
                trow[...] = jnp.zeros((16,), jnp.float32)
                grow[...] = jnp.zeros((16,), jnp.float32)
                x0row[...] = ch[pl.ds(0, 16)]

            def pass_a(i, carry):
                cm, t, g = carry
                v = ch[pl.ds(i * 16, 16)]
                cm = jnp.maximum(cm, v)
                t = t + v
                hit = iota == (y0v - (c * _CW + i * 16))
                g = g + jnp.where(hit, v, 0.0)
                return cm, t, g

            cm, t, g = jax.lax.fori_loop(
                0, _CW // 16, pass_a,
                (jnp.full((16,), -jnp.inf, jnp.float32),
                 trow[...], grow[...]))

            m_old = mrow[...]
            m_new = jnp.maximum(m_old, cm)
            s_base = srow[...] * jnp.exp(m_old - m_new)

            def pass_b(i, s):
                v = ch[pl.ds(i * 16, 16)]
                return s + jnp.exp(v - m_new)

            s = jax.lax.fori_loop(0, _CW // 16, pass_b, s_base)

            mrow[...] = m_new
            srow[...] = s
            trow[...] = t
            grow[...] = g

            orow = o_vmem.at[r8]
            orow[pl.ds(0, 16)] = m_new
            orow[pl.ds(16, 16)] = s
            orow[pl.ds(32, 16)] = t
            orow[pl.ds(48, 16)] = g
            orow[pl.ds(64, 16)] = x0row[...]

    pltpu.emit_pipeline(
        chunk_body,
        grid=(_SC_ROWS // 8, _NCH),
        in_specs=[
            pl.BlockSpec((8, _CW), lambda g, c: (row0 // 8 + g, c)),
            pl.BlockSpec((8, 16), lambda g, c: (g, 0)),
        ],
        out_specs=[pl.BlockSpec((8, 80), lambda g, c: (g, 0))],
        core_axis_name=("core", "subcore"),
        dimension_semantics=(pltpu.PARALLEL, pltpu.ARBITRARY),
    )(x_hbm, y_hbm, o_hbm)


@jax.jit
def kernel(x, y):
    B, C = x.shape
    tc_rows = B - _SC_ROWS
    sc_cols = _NCH * _CW
    cw_corner = C - sc_cols
    y2 = y.astype(jnp.int32).reshape(B, 1)

    ysc_b = jnp.tile(y2[tc_rows:], (1, 16))
    sc_mesh = plsc.VectorSubcoreMesh(core_axis_name="core",
                                     subcore_axis_name="subcore")
    sc_stats = pl.kernel(
        functools.partial(_sc_kernel, row0=tc_rows),
        out_type=jax.ShapeDtypeStruct((_SC_ROWS, 80), jnp.float32),
        mesh=sc_mesh,
        scratch_types=[pltpu.VMEM((8, 16), jnp.float32)] * 5,
    )(x, ysc_b)

    x_corner = jax.lax.slice(x, (tc_rows, sc_cols), (B, C))
    cm, cs, ct, cg = pl.pallas_call(
        functools.partial(_corner_kernel, CW=cw_corner, col0=sc_cols),
        grid=(_SC_ROWS // _RB,),
        in_specs=[
            pl.BlockSpec((_RB, cw_corner), lambda i: (i, 0)),
            pl.BlockSpec((_RB, 1), lambda i: (i, 0)),
        ],
        out_specs=[pl.BlockSpec((_RB, 1), lambda i: (i, 0))] * 4,
        out_shape=[jax.ShapeDtypeStruct((_SC_ROWS, 1), jnp.float32)] * 4,
    )(x_corner, y2[tc_rows:])

    def stripe_spec(q):
        return pl.BlockSpec((_RB, _W), lambda i, q=q: (i, q))

    rows = pl.pallas_call(
        functools.partial(_tc_kernel, C=C),
        grid=(tc_rows // _RB,),
        in_specs=[stripe_spec(q) for q in range(_NSTRIPE)]
        + [pl.BlockSpec((_RB, 1), lambda i: (i, 0))],
        out_specs=pl.BlockSpec((_RB, 1), lambda i: (i, 0)),
        out_shape=jax.ShapeDtypeStruct((tc_rows, 1), x.dtype),
        compiler_params=pltpu.CompilerParams(
            dimension_semantics=("arbitrary",),
        ),
    )(*([x] * _NSTRIPE), y2)

    eps = _SMOOTH / (C - 2)
    K = _SMOOTH * jnp.log(eps) + _CONF * jnp.log(_CONF)
    m16 = sc_stats[:, 0:16]
    s16 = sc_stats[:, 16:32]
    t16 = sc_stats[:, 32:48]
    g16 = sc_stats[:, 48:64]
    x0 = sc_stats[:, 64]
    m = jnp.maximum(jnp.max(m16, axis=1), cm[:, 0])
    s = (jnp.sum(s16 * jnp.exp(m16 - m[:, None]), axis=1)
         + cs[:, 0] * jnp.exp(cm[:, 0] - m))
    t = jnp.sum(t16, axis=1) + ct[:, 0]
    g = jnp.sum(g16, axis=1) + cg[:, 0]
    lse = m + jnp.log(s)
    ssum = t - C * lse
    logp0 = x0 - lse
    logpy = g - lse
    sc_row = K - eps * (ssum - logp0 - logpy) - _CONF * logpy
    y_sc = y[tc_rows:].astype(jnp.int32)
    sc_loss = jnp.sum(jnp.where(y_sc != _PAD, sc_row, 0.0))
    return jnp.sum(rows) + sc_loss


# speedup vs baseline: 1.3606x; 1.3606x over previous
import functools

import jax
import jax.numpy as jnp
from jax.experimental import pallas as pl
from jax.experimental.pallas import tpu as pltpu
from jax.experimental.pallas import tpu_sc as plsc

_SMOOTH = 0.1
_PAD = 0
_CONF = 1.0 - _SMOOTH

_NSTRIPE = 8
_RB = 32
_W = 12544

_SC_ROWS = 128
_CW = 3840
_NCH = 26


def _tc_kernel(*refs, C):
    xrefs = refs[:_NSTRIPE]
    y_ref = refs[_NSTRIPE]
    out_ref = refs[_NSTRIPE + 1]
    yb = y_ref[...]

    ms, ss, ts, gs = [], [], [], []
    x0 = None
    for q, xr in enumerate(xrefs):
        xq = xr[...]
        cols = jax.lax.broadcasted_iota(jnp.int32, xq.shape, 1)
        n_valid = C - q * _W
        if n_valid < _W:
            xv = jnp.where(cols < n_valid, xq, -jnp.inf)
            tq = jnp.sum(jnp.where(cols < n_valid, xq, 0.0), axis=1,
                         keepdims=True)
        else:
            xv = xq
            tq = jnp.sum(xq, axis=1, keepdims=True)
        mq = jnp.max(xv, axis=1, keepdims=True)
        ms.append(mq)
        ss.append(jnp.sum(jnp.exp(xv - mq), axis=1, keepdims=True))
        ts.append(tq)
        gs.append(jnp.sum(jnp.where(cols == yb - q * _W, xq, 0.0), axis=1,
                          keepdims=True))
        if q == 0:
            x0 = xq[:, 0:1]

    m = functools.reduce(jnp.maximum, ms)
    s = sum(sq * jnp.exp(mq - m) for sq, mq in zip(ss, ms))
    t = sum(ts)
    g = sum(gs)

    eps = _SMOOTH / (C - 2)
    K = _SMOOTH * jnp.log(eps) + _CONF * jnp.log(_CONF)
    lse = m + jnp.log(s)
    ssum = t - C * lse
    logp0 = x0 - lse
    logpy = g - lse
    row = K - eps * (ssum - logp0 - logpy) - _CONF * logpy
    out_ref[...] = jnp.where(yb != _PAD, row, 0.0)


def _corner_kernel(x_ref, y_ref, m_ref, s_ref, t_ref, g_ref, *, CW, col0):
    xq = x_ref[...]
    yb = y_ref[...]
    cols = jax.lax.broadcasted_iota(jnp.int32, xq.shape, 1)
    valid = cols < CW
    xv = jnp.where(valid, xq, -jnp.inf)
    mq = jnp.max(xv, axis=1, keepdims=True)
    m_ref[...] = mq
    s_ref[...] = jnp.sum(jnp.exp(xv - mq), axis=1, keepdims=True)
    t_ref[...] = jnp.sum(jnp.where(valid, xq, 0.0), axis=1, keepdims=True)
    g_ref[...] = jnp.sum(jnp.where(cols == yb - col0, xq, 0.0), axis=1,
                         keepdims=True)


def _sc_kernel(x_hbm, y_hbm, o_hbm, m_s, s_s, t_s, g_s, x0_s, *, row0):
    iota = jax.lax.iota(jnp.int32, 16)

    def chunk_body(x_vmem, y_vmem, o_vmem):
        c = pl.program_id(1)
        for r8 in range(8):
            ch = x_vmem.at[r8]
            y0v = y_vmem.at[r8][pl.ds(0, 16)]
            mrow = m_s.at[r8]
            srow = s_s.at[r8]
            trow = t_s.at[r8]
            grow = g_s.at[r8]
            x0row = x0_s.at[r8]

            @pl.when(c == 0)
            def _init():
                mrow[...] = jnp.full((16,), -jnp.inf, jnp.float32)
                srow[...] = jnp.zeros((16,), jnp.float32)
                trow[...] = jnp.zeros((16,), jnp.float32)
                grow[...] = jnp.zeros((16,), jnp.float32)
                x0row[...] = ch[pl.ds(0, 16)]

            def pass_a(i, carry):
                cm, t, g = carry
                v = ch[pl.ds(i * 16, 16)]
                cm = jnp.maximum(cm, v)
                t = t + v
                hit = iota == (y0v - (c * _CW + i * 16))
                g = g + jnp.where(hit, v, 0.0)
                return cm, t, g

            cm, t, g = jax.lax.fori_loop(
                0, _CW // 16, pass_a,
                (jnp.full((16,), -jnp.inf, jnp.float32),
                 trow[...], grow[...]))

            m_old = mrow[...]
            m_new = jnp.maximum(m_old, cm)
            s_base = srow[...] * jnp.exp(m_old - m_new)

            def pass_b(i, s):
                v = ch[pl.ds(i * 16, 16)]
                return s + jnp.exp(v - m_new)

            s = jax.lax.fori_loop(0, _CW // 16, pass_b, s_base)

            mrow[...] = m_new
            srow[...] = s
            trow[...] = t
            grow[...] = g

            orow = o_vmem.at[r8]
            orow[pl.ds(0, 16)] = m_new
            orow[pl.ds(16, 16)] = s
            orow[pl.ds(32, 16)] = t
            orow[pl.ds(48, 16)] = g
            orow[pl.ds(64, 16)] = x0row[...]

    pltpu.emit_pipeline(
        chunk_body,
        grid=(_SC_ROWS // 8, _NCH),
        in_specs=[
            pl.BlockSpec((8, _CW), lambda g, c: (row0 // 8 + g, c)),
            pl.BlockSpec((8, 16), lambda g, c: (g, 0)),
        ],
        out_specs=[pl.BlockSpec((8, 80), lambda g, c: (g, 0))],
        core_axis_name=("core", "subcore"),
        dimension_semantics=(pltpu.PARALLEL, pltpu.ARBITRARY),
    )(x_hbm, y_hbm, o_hbm)


@jax.jit
def kernel(x, y):
    B, C = x.shape
    tc_rows = B - _SC_ROWS
    sc_cols = _NCH * _CW
    cw_corner = C - sc_cols
    y2 = y.astype(jnp.int32).reshape(B, 1)

    ysc_b = jnp.tile(y2[tc_rows:], (1, 16))
    sc_mesh = plsc.VectorSubcoreMesh(core_axis_name="core",
                                     subcore_axis_name="subcore")
    sc_stats = pl.kernel(
        functools.partial(_sc_kernel, row0=tc_rows),
        out_type=jax.ShapeDtypeStruct((_SC_ROWS, 80), jnp.float32),
        mesh=sc_mesh,
        scratch_types=[pltpu.VMEM((8, 16), jnp.float32)] * 5,
    )(x, ysc_b)

    x_corner = jax.lax.slice(x, (tc_rows, sc_cols), (B, C))
    cm, cs, ct, cg = pl.pallas_call(
        functools.partial(_corner_kernel, CW=cw_corner, col0=sc_cols),
        grid=(_SC_ROWS // _RB,),
        in_specs=[
            pl.BlockSpec((_RB, cw_corner), lambda i: (i, 0)),
            pl.BlockSpec((_RB, 1), lambda i: (i, 0)),
        ],
        out_specs=[pl.BlockSpec((_RB, 1), lambda i: (i, 0))] * 4,
        out_shape=[jax.ShapeDtypeStruct((_SC_ROWS, 1), jnp.float32)] * 4,
    )(x_corner, y2[tc_rows:])

    def stripe_spec(q):
        return pl.BlockSpec((_RB, _W), lambda i, q=q: (i, q))

    rows = pl.pallas_call(
        functools.partial(_tc_kernel, C=C),
        grid=(tc_rows // _RB,),
        in_specs=[stripe_spec(q) for q in range(_NSTRIPE)]
        + [pl.BlockSpec((_RB, 1), lambda i: (i, 0))],
        out_specs=pl.BlockSpec((_RB, 1), lambda i: (i, 0)),
        out_shape=jax.ShapeDtypeStruct((tc_rows, 1), x.dtype),
        compiler_params=pltpu.CompilerParams(
            dimension_semantics=("arbitrary",),
        ),
    )(*([x] * _NSTRIPE), y2)

    eps = _SMOOTH / (C - 2)
    K = _SMOOTH * jnp.log(eps) + _CONF * jnp.log(_CONF)
    m16 = sc_stats[:, 0:16]
    s16 = sc_stats[:, 16:32]
    t16 = sc_stats[:, 32:48]
    g16 = sc_stats[:, 48:64]
    x0 = sc_stats[:, 64]
    m = jnp.maximum(jnp.max(m16, axis=1), cm[:, 0])
    s = (jnp.sum(s16 * jnp.exp(m16 - m[:, None]), axis=1)
         + cs[:, 0] * jnp.exp(cm[:, 0] - m))
    t = jnp.sum(t16, axis=1) + ct[:, 0]
    g = jnp.sum(g16, axis=1) + cg[:, 0]
    lse = m + jnp.log(s)
    ssum = t - C * lse
    logp0 = x0 - lse
    logpy = g - lse
    sc_row = K - eps * (ssum - logp0 - logpy) - _CONF * logpy
    y_sc = y[tc_rows:].astype(jnp.int32)
    sc_loss = jnp.sum(jnp.where(y_sc != _PAD, sc_row, 0.0))
    return jnp.sum(rows) + sc_loss


# speedup vs baseline: 1.5639x; 1.1494x over previous
import functools

import jax
import jax.numpy as jnp
from jax.experimental import pallas as pl
from jax.experimental.pallas import tpu as pltpu

_SMOOTH = 0.1
_PAD = 0
_CONF = 1.0 - _SMOOTH
_NSTRIPE = 8


def _rowloss_kernel(*refs, C, W):
    xrefs = refs[:_NSTRIPE]
    y_ref = refs[_NSTRIPE]
    out_ref = refs[_NSTRIPE + 1]
    yb = y_ref[...]

    ms, ss, ts, gs = [], [], [], []
    x0 = None
    for q, xr in enumerate(xrefs):
        xq = xr[...]
        cols = jax.lax.broadcasted_iota(jnp.int32, xq.shape, 1)
        n_valid = C - q * W
        if n_valid < W:
            xv = jnp.where(cols < n_valid, xq, -jnp.inf)
            tq = jnp.sum(jnp.where(cols < n_valid, xq, 0.0), axis=1,
                         keepdims=True)
        else:
            xv = xq
            tq = jnp.sum(xq, axis=1, keepdims=True)
        mq = jnp.max(xv, axis=1, keepdims=True)
        ms.append(mq)
        ss.append(jnp.sum(jnp.exp(xv - mq), axis=1, keepdims=True))
        ts.append(tq)
        gs.append(jnp.sum(jnp.where(cols == yb - q * W, xq, 0.0), axis=1,
                          keepdims=True))
        if q == 0:
            x0 = xq[:, 0:1]

    m = functools.reduce(jnp.maximum, ms)
    s = sum(sq * jnp.exp(mq - m) for sq, mq in zip(ss, ms))
    t = sum(ts)
    g = sum(gs)

    eps = _SMOOTH / (C - 2)
    K = _SMOOTH * jnp.log(eps) + _CONF * jnp.log(_CONF)
    lse = m + jnp.log(s)
    ssum = t - C * lse
    logp0 = x0 - lse
    logpy = g - lse
    row = K - eps * (ssum - logp0 - logpy) - _CONF * logpy
    out_ref[...] = jnp.where(yb != _PAD, row, 0.0)


@jax.jit
def kernel(x, y):
    B, C = x.shape
    Rb = 64
    n_rb = B // Rb
    W = 12544
    y2 = y.astype(jnp.int32).reshape(B, 1)

    def stripe_spec(q):
        return pl.BlockSpec((Rb, W), lambda i, q=q: (i, q))

    rows = pl.pallas_call(
        functools.partial(_rowloss_kernel, C=C, W=W),
        grid=(n_rb,),
        in_specs=[stripe_spec(q) for q in range(_NSTRIPE)]
        + [pl.BlockSpec((Rb, 1), lambda i: (i, 0))],
        out_specs=pl.BlockSpec((Rb, 1), lambda i: (i, 0)),
        out_shape=jax.ShapeDtypeStruct((B, 1), x.dtype),
        compiler_params=pltpu.CompilerParams(
            dimension_semantics=("arbitrary",),
        ),
    )(*([x] * _NSTRIPE), y2)
    return jnp.sum(rows)
